# baseline (speedup 1.0000x reference)
import functools

import jax
import jax.numpy as jnp
from jax import lax
from jax.experimental import pallas as pl
from jax.experimental.pallas import tpu as pltpu
from jax.experimental.pallas import tpu_sc as plsc

U, W, NUM_CATS, B = 16, 16, 100000, 4096
R = U * W
NC, NS = 2, 16
NW = NC * NS
RPW = R // NW


def _sc_gather(A2d, assignment):
  mesh = plsc.VectorSubcoreMesh(core_axis_name="c", subcore_axis_name="s")

  @functools.partial(
      pl.kernel,
      out_type=jax.ShapeDtypeStruct((R, B), jnp.float32),
      mesh=mesh,
      scratch_types=[
          pltpu.VMEM((B,), jnp.int32),
          pltpu.VMEM((NUM_CATS,), jnp.float32),
          pltpu.VMEM((B,), jnp.float32),
      ],
      compiler_params=pltpu.CompilerParams(needs_layout_passes=False),
  )
  def gather_kernel(a_hbm, asg_hbm, g_hbm, asg_v, row_v, grow_v):
    wid = lax.axis_index("c") * NS + lax.axis_index("s")
    row0 = wid * RPW
    pltpu.sync_copy(asg_hbm, asg_v)
    for r in range(RPW):
      pltpu.sync_copy(a_hbm.at[row0 + r], row_v)

      def body(i, _):
        for j in range(8):
          off = i * 128 + j * 16
          idx = asg_v[pl.ds(off, 16)]
          grow_v[pl.ds(off, 16)] = plsc.load_gather(row_v, [idx])
        return 0

      lax.fori_loop(0, B // 128, body, 0)
      pltpu.sync_copy(grow_v, g_hbm.at[row0 + r])

  return gather_kernel(A2d, assignment)


def _tc_expand(G3):
  BB = 1024

  def body(g_ref, o_ref):
    g = g_ref[0]
    o_ref[0] = g[:, None, :] + g[None, :, :]

  return pl.pallas_call(
      body,
      grid=(U, B // BB),
      in_specs=[pl.BlockSpec((1, W, BB), lambda u, b: (u, 0, b))],
      out_specs=pl.BlockSpec((1, W, W, BB), lambda u, b: (u, 0, 0, b)),
      out_shape=jax.ShapeDtypeStruct((U, W, W, B), jnp.float32),
  )(G3)


@jax.jit
def kernel(A, assignment):
  A2d = A.reshape(R, NUM_CATS)
  G = _sc_gather(A2d, assignment)
  return _tc_expand(G.reshape(U, W, B))


# speedup vs baseline: 1.2589x; 1.2589x over previous
import functools

import jax
import jax.numpy as jnp
from jax import lax
from jax.experimental import pallas as pl
from jax.experimental.pallas import tpu as pltpu
from jax.experimental.pallas import tpu_sc as plsc

U, W, NUM_CATS, B = 16, 16, 100000, 4096
R = U * W
NC, NS = 2, 16
NW = NC * NS
RPW = R // NW


def _sc_gather(A2d, assignment):
  mesh = plsc.VectorSubcoreMesh(core_axis_name="c", subcore_axis_name="s")

  @functools.partial(
      pl.kernel,
      out_type=jax.ShapeDtypeStruct((R, B), jnp.float32),
      mesh=mesh,
      scratch_types=[
          pltpu.VMEM((B,), jnp.int32),
          pltpu.VMEM((NUM_CATS,), jnp.float32),
          pltpu.VMEM((B,), jnp.float32),
      ],
      compiler_params=pltpu.CompilerParams(needs_layout_passes=False),
  )
  def gather_kernel(a_hbm, asg_hbm, g_hbm, asg_v, row_v, grow_v):
    wid = lax.axis_index("c") * NS + lax.axis_index("s")
    row0 = wid * RPW
    pltpu.sync_copy(asg_hbm, asg_v)
    for r in range(RPW):
      pltpu.sync_copy(a_hbm.at[row0 + r], row_v)

      def body(i, _):
        for j in range(8):
          off = i * 128 + j * 16
          idx = asg_v[pl.ds(off, 16)]
          grow_v[pl.ds(off, 16)] = plsc.load_gather(row_v, [idx])
        return 0

      lax.fori_loop(0, B // 128, body, 0)
      pltpu.sync_copy(grow_v, g_hbm.at[row0 + r])

  return gather_kernel(A2d, assignment)


def _tc_expand(G3):
  UB = 2

  def body(g_ref, o_ref):
    for k in range(UB):
      g = g_ref[k]
      o_ref[k] = g[:, None, :] + g[None, :, :]

  return pl.pallas_call(
      body,
      grid=(U // UB,),
      in_specs=[pl.BlockSpec((UB, W, B), lambda u: (u, 0, 0))],
      out_specs=pl.BlockSpec((UB, W, W, B), lambda u: (u, 0, 0, 0)),
      out_shape=jax.ShapeDtypeStruct((U, W, W, B), jnp.float32),
  )(G3)


@jax.jit
def kernel(A, assignment):
  A2d = A.reshape(R, NUM_CATS)
  G = _sc_gather(A2d, assignment)
  return _tc_expand(G.reshape(U, W, B))
